# baseline (speedup 1.0000x reference)
import functools

import jax
import jax.numpy as jnp
from jax import lax
from jax.experimental import pallas as pl
from jax.experimental.pallas import tpu as pltpu
from jax.experimental.pallas import tpu_sc as plsc

_MAX_DAYS = 3650

_NC = 2
_NS = 16
_NW = _NC * _NS


def _sc_gather(pe, idx, rows_per_w=16):
    t = idx.shape[0]
    d = pe.shape[1]
    nw_full = t // rows_per_w
    rem = t - nw_full * rows_per_w
    max_row = pe.shape[0] - 1

    mesh = plsc.VectorSubcoreMesh(
        core_axis_name="c", subcore_axis_name="s", num_cores=1)

    @functools.partial(
        pl.kernel,
        mesh=mesh,
        out_type=jax.ShapeDtypeStruct((t, d), jnp.float32),
        scratch_types=[
            pltpu.VMEM((rows_per_w,), jnp.int32),
            pltpu.VMEM((rows_per_w, d), jnp.float32),
            pltpu.SemaphoreType.DMA,
        ],
    )
    def gather_kernel(pe_hbm, idx_hbm, out_hbm, idx_v, rows_v, sem):
        wid = lax.axis_index("s")

        @pl.when(wid <= nw_full)
        def _():
            base = jnp.minimum(wid * rows_per_w, t - rows_per_w)
            pltpu.sync_copy(idx_hbm.at[pl.ds(base, rows_per_w)], idx_v)
            idx_v[...] = jnp.minimum(idx_v[...], max_row)
            pltpu.async_copy(pe_hbm.at[idx_v], rows_v, sem).wait()

            @pl.when(wid < nw_full)
            def _():
                pltpu.sync_copy(rows_v, out_hbm.at[pl.ds(base, rows_per_w)])

            if rem:
                @pl.when(wid == nw_full)
                def _():
                    pltpu.sync_copy(
                        rows_v.at[pl.ds(rows_per_w - rem, rem)],
                        out_hbm.at[pl.ds(t - rem, rem)])

    return gather_kernel(pe, idx)


def _mlp(dn_ref, w1_ref, b1_ref, w2_ref, b2_ref):
    dn_row = jnp.minimum(dn_ref[...], _MAX_DAYS).astype(jnp.float32) / _MAX_DAYS
    dn = lax.transpose(dn_row, (1, 0))
    h = dn * w1_ref[...] + b1_ref[...]
    h = h * jax.nn.sigmoid(h)
    lp = jnp.dot(h, w2_ref[...], preferred_element_type=jnp.float32,
                 precision=lax.Precision.HIGHEST)
    return lp + b2_ref[...]


def _tc_body_a(dn_ref, ds_ref, w1_ref, b1_ref, w2_ref, b2_ref, pe_ref, emb_ref,
               out_ref, add_ref):
    @pl.when(pl.program_id(0) == 0)
    def _():
        t = add_ref.shape[0]

        def row(tt, _):
            dd = jnp.minimum(ds_ref[0, tt], _MAX_DAYS)
            add_ref[pl.ds(tt, 1), :] = pe_ref[pl.ds(dd, 1), :]
            return 0

        lax.fori_loop(0, t, row, 0)
        add_ref[...] = add_ref[...] + _mlp(dn_ref, w1_ref, b1_ref, w2_ref,
                                           b2_ref)

    out_ref[...] = emb_ref[...] + add_ref[...][None, :, :]


def _tc_body_b(canvas_ref, dn_ref, w1_ref, b1_ref, w2_ref, b2_ref, sin_ref,
               emb_ref, out_ref, add_ref):
    del canvas_ref
    @pl.when(pl.program_id(0) == 0)
    def _():
        add_ref[...] = sin_ref[...] + _mlp(dn_ref, w1_ref, b1_ref, w2_ref,
                                           b2_ref)

    out_ref[...] = emb_ref[...] + add_ref[...][None, :, :]


def kernel(embeddings, days_since_baseline, pe, W1, b1, W2, b2):
    b, t, d = embeddings.shape
    d4 = W1.shape[1]

    days = days_since_baseline.astype(jnp.int32)
    sin_pe = _sc_gather(pe, days)

    days_row = days[None, :]
    b1r = b1[None, :]
    b2r = b2[None, :]

    bb = 128
    n_blocks = b // bb
    n_head = 1
    n_tail = n_blocks - n_head

    small_specs = [
        pl.BlockSpec((1, t), lambda i: (0, 0)),
        pl.BlockSpec((1, d4), lambda i: (0, 0)),
        pl.BlockSpec((1, d4), lambda i: (0, 0)),
        pl.BlockSpec((d4, d), lambda i: (0, 0)),
        pl.BlockSpec((1, d), lambda i: (0, 0)),
    ]

    head = pl.pallas_call(
        _tc_body_a,
        grid=(n_head,),
        in_specs=small_specs[:1] + [
            pl.BlockSpec((1, t), lambda i: (0, 0),
                         memory_space=pltpu.SMEM),
        ] + small_specs[1:] + [
            pl.BlockSpec(pe.shape, lambda i: (0, 0)),
            pl.BlockSpec((bb, t, d), lambda i: (i, 0, 0)),
        ],
        out_specs=pl.BlockSpec((bb, t, d), lambda i: (i, 0, 0)),
        out_shape=jax.ShapeDtypeStruct((b, t, d), jnp.float32),
        scratch_shapes=[pltpu.VMEM((t, d), jnp.float32)],
    )(days_row, days_row, W1, b1r, W2, b2r, pe, embeddings)

    out = pl.pallas_call(
        _tc_body_b,
        grid=(n_tail,),
        in_specs=[pl.BlockSpec(memory_space=pl.ANY)] + small_specs + [
            pl.BlockSpec((t, d), lambda i: (0, 0)),
            pl.BlockSpec((bb, t, d), lambda i: (i + n_head, 0, 0)),
        ],
        out_specs=pl.BlockSpec((bb, t, d), lambda i: (i + n_head, 0, 0)),
        out_shape=jax.ShapeDtypeStruct((b, t, d), jnp.float32),
        scratch_shapes=[pltpu.VMEM((t, d), jnp.float32)],
        input_output_aliases={0: 0},
    )(head, days_row, W1, b1r, W2, b2r, sin_pe, embeddings)
    return out


# speedup vs baseline: 1.0344x; 1.0344x over previous
import functools

import jax
import jax.numpy as jnp
from jax import lax
from jax.experimental import pallas as pl
from jax.experimental.pallas import tpu as pltpu
from jax.experimental.pallas import tpu_sc as plsc

_MAX_DAYS = 3650

_NC = 2
_NS = 16
_NW = _NC * _NS


def _sc_gather(pe, idx, rows_per_w=16):
    t = idx.shape[0]
    d = pe.shape[1]
    nw_full = t // rows_per_w
    rem = t - nw_full * rows_per_w
    max_row = pe.shape[0] - 1

    mesh = plsc.VectorSubcoreMesh(
        core_axis_name="c", subcore_axis_name="s", num_cores=1)

    @functools.partial(
        pl.kernel,
        mesh=mesh,
        out_type=jax.ShapeDtypeStruct((t, d), jnp.float32),
        scratch_types=[
            pltpu.VMEM((rows_per_w,), jnp.int32),
            pltpu.VMEM((rows_per_w, d), jnp.float32),
            pltpu.SemaphoreType.DMA,
        ],
    )
    def gather_kernel(pe_hbm, idx_hbm, out_hbm, idx_v, rows_v, sem):
        wid = lax.axis_index("s")

        @pl.when(wid <= nw_full)
        def _():
            base = jnp.minimum(wid * rows_per_w, t - rows_per_w)
            pltpu.sync_copy(idx_hbm.at[pl.ds(base, rows_per_w)], idx_v)
            idx_v[...] = jnp.minimum(idx_v[...], max_row)
            pltpu.async_copy(pe_hbm.at[idx_v], rows_v, sem).wait()

            @pl.when(wid < nw_full)
            def _():
                pltpu.sync_copy(rows_v, out_hbm.at[pl.ds(base, rows_per_w)])

            if rem:
                @pl.when(wid == nw_full)
                def _():
                    pltpu.sync_copy(
                        rows_v.at[pl.ds(rows_per_w - rem, rem)],
                        out_hbm.at[pl.ds(t - rem, rem)])

    return gather_kernel(pe, idx)


def _mlp(dn_ref, w1_ref, b1_ref, w2_ref, b2_ref):
    dn_row = jnp.minimum(dn_ref[...], _MAX_DAYS).astype(jnp.float32) / _MAX_DAYS
    dn = lax.transpose(dn_row, (1, 0))
    h = dn * w1_ref[...] + b1_ref[...]
    h = h * jax.nn.sigmoid(h)
    lp = jnp.dot(h, w2_ref[...], preferred_element_type=jnp.float32,
                 precision=lax.Precision.HIGHEST)
    return lp + b2_ref[...]


def _tc_body_a(dn_ref, ds_ref, w1_ref, b1_ref, w2_ref, b2_ref, pe_ref, emb_ref,
               out_ref, add_ref):
    @pl.when(pl.program_id(0) == 0)
    def _():
        t = add_ref.shape[0]

        def row(tt, _):
            dd = jnp.minimum(ds_ref[0, tt], _MAX_DAYS)
            add_ref[pl.ds(tt, 1), :] = pe_ref[pl.ds(dd, 1), :]
            return 0

        lax.fori_loop(0, t, row, 0)
        add_ref[...] = add_ref[...] + _mlp(dn_ref, w1_ref, b1_ref, w2_ref,
                                           b2_ref)

    out_ref[...] = emb_ref[...] + add_ref[...][None, :, :]


def _tc_body_b(canvas_ref, dn_ref, w1_ref, b1_ref, w2_ref, b2_ref, sin_ref,
               emb_ref, out_ref, add_ref):
    del canvas_ref
    @pl.when(pl.program_id(0) == 0)
    def _():
        add_ref[...] = sin_ref[...] + _mlp(dn_ref, w1_ref, b1_ref, w2_ref,
                                           b2_ref)

    out_ref[...] = emb_ref[...] + add_ref[...][None, :, :]


def kernel(embeddings, days_since_baseline, pe, W1, b1, W2, b2):
    b, t, d = embeddings.shape
    d4 = W1.shape[1]

    days = days_since_baseline.astype(jnp.int32)
    sin_pe = _sc_gather(pe, days)

    days_row = days[None, :]
    b1r = b1[None, :]
    b2r = b2[None, :]

    bb = 128
    n_blocks = b // bb
    n_head = 2
    n_tail = n_blocks - n_head

    small_specs = [
        pl.BlockSpec((1, t), lambda i: (0, 0)),
        pl.BlockSpec((1, d4), lambda i: (0, 0)),
        pl.BlockSpec((1, d4), lambda i: (0, 0)),
        pl.BlockSpec((d4, d), lambda i: (0, 0)),
        pl.BlockSpec((1, d), lambda i: (0, 0)),
    ]

    head = pl.pallas_call(
        _tc_body_a,
        grid=(n_head,),
        in_specs=small_specs[:1] + [
            pl.BlockSpec((1, t), lambda i: (0, 0),
                         memory_space=pltpu.SMEM),
        ] + small_specs[1:] + [
            pl.BlockSpec(pe.shape, lambda i: (0, 0)),
            pl.BlockSpec((bb, t, d), lambda i: (i, 0, 0)),
        ],
        out_specs=pl.BlockSpec((bb, t, d), lambda i: (i, 0, 0)),
        out_shape=jax.ShapeDtypeStruct((b, t, d), jnp.float32),
        scratch_shapes=[pltpu.VMEM((t, d), jnp.float32)],
    )(days_row, days_row, W1, b1r, W2, b2r, pe, embeddings)

    out = pl.pallas_call(
        _tc_body_b,
        grid=(n_tail,),
        in_specs=[pl.BlockSpec(memory_space=pl.ANY)] + small_specs + [
            pl.BlockSpec((t, d), lambda i: (0, 0)),
            pl.BlockSpec((bb, t, d), lambda i: (i + n_head, 0, 0)),
        ],
        out_specs=pl.BlockSpec((bb, t, d), lambda i: (i + n_head, 0, 0)),
        out_shape=jax.ShapeDtypeStruct((b, t, d), jnp.float32),
        scratch_shapes=[pltpu.VMEM((t, d), jnp.float32)],
        input_output_aliases={0: 0},
    )(head, days_row, W1, b1r, W2, b2r, sin_pe, embeddings)
    return out


# speedup vs baseline: 1.0402x; 1.0056x over previous
import functools

import jax
import jax.numpy as jnp
from jax import lax
from jax.experimental import pallas as pl
from jax.experimental.pallas import tpu as pltpu
from jax.experimental.pallas import tpu_sc as plsc

_MAX_DAYS = 3650


def _sc_gather(pe, idx, rows_per_w=16):
    t = idx.shape[0]
    d = pe.shape[1]
    nw_full = t // rows_per_w
    rem = t - nw_full * rows_per_w
    max_row = pe.shape[0] - 1

    mesh = plsc.VectorSubcoreMesh(
        core_axis_name="c", subcore_axis_name="s", num_cores=1)

    @functools.partial(
        pl.kernel,
        mesh=mesh,
        out_type=jax.ShapeDtypeStruct((t, d), jnp.float32),
        scratch_types=[
            pltpu.VMEM((rows_per_w,), jnp.int32),
            pltpu.VMEM((rows_per_w, d), jnp.float32),
            pltpu.SemaphoreType.DMA,
        ],
    )
    def gather_kernel(pe_hbm, idx_hbm, out_hbm, idx_v, rows_v, sem):
        wid = lax.axis_index("s")

        @pl.when(wid <= nw_full)
        def _():
            base = jnp.minimum(wid * rows_per_w, t - rows_per_w)
            pltpu.sync_copy(idx_hbm.at[pl.ds(base, rows_per_w)], idx_v)
            idx_v[...] = jnp.minimum(idx_v[...], max_row)
            pltpu.async_copy(pe_hbm.at[idx_v], rows_v, sem).wait()

            @pl.when(wid < nw_full)
            def _():
                pltpu.sync_copy(rows_v, out_hbm.at[pl.ds(base, rows_per_w)])

            if rem:
                @pl.when(wid == nw_full)
                def _():
                    pltpu.sync_copy(
                        rows_v.at[pl.ds(rows_per_w - rem, rem)],
                        out_hbm.at[pl.ds(t - rem, rem)])

    return gather_kernel(pe, idx)


def _mlp(dn_ref, w1_ref, b1_ref, w2_ref, b2_ref):
    dn_row = jnp.minimum(dn_ref[...], _MAX_DAYS).astype(jnp.float32) / _MAX_DAYS
    dn = lax.transpose(dn_row, (1, 0))
    h = dn * w1_ref[...] + b1_ref[...]
    h = h * jax.nn.sigmoid(h)
    lp = jnp.dot(h, w2_ref[...], preferred_element_type=jnp.float32,
                 precision=lax.Precision.HIGHEST)
    return lp + b2_ref[...]


def _tc_body_a(dn_ref, ds_ref, w1_ref, b1_ref, w2_ref, b2_ref, pe_ref, emb_ref,
               out_ref, add_ref):
    @pl.when(pl.program_id(0) == 0)
    def _():
        t = add_ref.shape[0]

        def row(tt, _):
            dd = jnp.minimum(ds_ref[0, tt], _MAX_DAYS)
            add_ref[pl.ds(tt, 1), :] = pe_ref[pl.ds(dd, 1), :]
            return 0

        lax.fori_loop(0, t, row, 0)
        add_ref[...] = add_ref[...] + _mlp(dn_ref, w1_ref, b1_ref, w2_ref,
                                           b2_ref)

    out_ref[...] = emb_ref[...] + add_ref[...][None, :, :]


def _tc_body_b(canvas_ref, dn_ref, w1_ref, b1_ref, w2_ref, b2_ref, sin_ref,
               emb_ref, out_ref, add_ref):
    del canvas_ref
    @pl.when(pl.program_id(0) == 0)
    def _():
        add_ref[...] = sin_ref[...] + _mlp(dn_ref, w1_ref, b1_ref, w2_ref,
                                           b2_ref)

    out_ref[...] = emb_ref[...] + add_ref[...][None, :, :]


def kernel(embeddings, days_since_baseline, pe, W1, b1, W2, b2):
    b, t, d = embeddings.shape
    d4 = W1.shape[1]

    days = days_since_baseline.astype(jnp.int32)
    sin_pe = _sc_gather(pe, days)

    days_row = days[None, :]
    b1r = b1[None, :]
    b2r = b2[None, :]

    bb = 128
    n_blocks = b // bb
    n_head = 2
    n_tail = n_blocks - n_head

    small_specs = [
        pl.BlockSpec((1, t), lambda i: (0, 0)),
        pl.BlockSpec((1, d4), lambda i: (0, 0)),
        pl.BlockSpec((1, d4), lambda i: (0, 0)),
        pl.BlockSpec((d4, d), lambda i: (0, 0)),
        pl.BlockSpec((1, d), lambda i: (0, 0)),
    ]

    head = pl.pallas_call(
        _tc_body_a,
        grid=(n_head,),
        in_specs=small_specs[:1] + [
            pl.BlockSpec((1, t), lambda i: (0, 0),
                         memory_space=pltpu.SMEM),
        ] + small_specs[1:] + [
            pl.BlockSpec(pe.shape, lambda i: (0, 0)),
            pl.BlockSpec((bb, t, d), lambda i: (i, 0, 0)),
        ],
        out_specs=pl.BlockSpec((bb, t, d), lambda i: (i, 0, 0)),
        out_shape=jax.ShapeDtypeStruct((b, t, d), jnp.float32),
        scratch_shapes=[pltpu.VMEM((t, d), jnp.float32)],
    )(days_row, days_row, W1, b1r, W2, b2r, pe, embeddings)

    out = pl.pallas_call(
        _tc_body_b,
        grid=(n_tail,),
        in_specs=[pl.BlockSpec(memory_space=pl.ANY)] + small_specs + [
            pl.BlockSpec((t, d), lambda i: (0, 0)),
            pl.BlockSpec((bb, t, d), lambda i: (i + n_head, 0, 0)),
        ],
        out_specs=pl.BlockSpec((bb, t, d), lambda i: (i + n_head, 0, 0)),
        out_shape=jax.ShapeDtypeStruct((b, t, d), jnp.float32),
        scratch_shapes=[pltpu.VMEM((t, d), jnp.float32)],
        input_output_aliases={0: 0},
    )(head, days_row, W1, b1r, W2, b2r, sin_pe, embeddings)
    return out
